# initial kernel scaffold (unmeasured)
import jax
import jax.numpy as jnp
from jax import lax
from jax.experimental import pallas as pl
from jax.experimental.pallas import tpu as pltpu


def kernel(
    u,
):
    def body(*refs):
        pass

    out_shape = jax.ShapeDtypeStruct(..., jnp.float32)
    return pl.pallas_call(body, out_shape=out_shape)(...)


# baseline (device time: 8642 ns/iter reference)
import jax
import jax.numpy as jnp
from jax import lax
from jax.experimental import pallas as pl
from jax.experimental.pallas import tpu as pltpu


def kernel(u):
    s = u.shape[0]
    f32 = jnp.float32

    def body(u_ref, out_ref, send_buf, halo_buf, send_sems, recv_sems):
        px = lax.axis_index("x")
        py = lax.axis_index("y")
        pz = lax.axis_index("z")

        nbrs = [(1 - px, py, pz), (px, 1 - py, pz), (px, py, 1 - pz)]
        barrier_sem = pltpu.get_barrier_semaphore()
        for nbr in nbrs:
            pl.semaphore_signal(
                barrier_sem, inc=1,
                device_id=nbr, device_id_type=pl.DeviceIdType.MESH,
            )
        pl.semaphore_wait(barrier_sem, 3)

        uu = u_ref[...]

        send_buf[0] = jnp.where(px == 0, uu[s - 1, :, :], uu[0, :, :])
        send_buf[1] = jnp.where(py == 0, uu[:, s - 1, :], uu[:, 0, :])
        send_buf[2] = jnp.where(pz == 0, uu[:, :, s - 1], uu[:, :, 0])

        rdmas = []
        for a, nbr in enumerate(nbrs):
            rdma = pltpu.make_async_remote_copy(
                src_ref=send_buf.at[a],
                dst_ref=halo_buf.at[a],
                send_sem=send_sems.at[a],
                recv_sem=recv_sems.at[a],
                device_id=nbr,
                device_id_type=pl.DeviceIdType.MESH,
            )
            rdma.start()
            rdmas.append(rdma)
        for rdma in rdmas:
            rdma.wait()

        zero2 = jnp.zeros((s, s), f32)
        hx = halo_buf[0]
        hy = halo_buf[1]
        hz = halo_buf[2]

        x_lo = jnp.where(px == 1, hx, zero2)[None, :, :]
        x_hi = jnp.where(px == 0, hx, zero2)[None, :, :]
        y_lo = jnp.where(py == 1, hy, zero2)[:, None, :]
        y_hi = jnp.where(py == 0, hy, zero2)[:, None, :]
        z_lo = jnp.where(pz == 1, hz, zero2)[:, :, None]
        z_hi = jnp.where(pz == 0, hz, zero2)[:, :, None]

        u_xm = jnp.concatenate([x_lo, uu[:-1, :, :]], axis=0)
        u_xp = jnp.concatenate([uu[1:, :, :], x_hi], axis=0)
        u_ym = jnp.concatenate([y_lo, uu[:, :-1, :]], axis=1)
        u_yp = jnp.concatenate([uu[:, 1:, :], y_hi], axis=1)
        u_zm = jnp.concatenate([z_lo, uu[:, :, :-1]], axis=2)
        u_zp = jnp.concatenate([uu[:, :, 1:], z_hi], axis=2)

        v = u_xm + u_xp + u_ym + u_yp + u_zm + u_zp - 6.0 * uu

        gi = lax.broadcasted_iota(jnp.int32, (s, s, s), 0) + px * s
        gj = lax.broadcasted_iota(jnp.int32, (s, s, s), 1) + py * s
        gk = lax.broadcasted_iota(jnp.int32, (s, s, s), 2) + pz * s
        n = 2 * s - 1
        interior = (
            (gi > 0) & (gi < n) & (gj > 0) & (gj < n) & (gk > 0) & (gk < n)
        )
        out_ref[...] = jnp.where(interior, v, 0.0)

    return pl.pallas_call(
        body,
        out_shape=jax.ShapeDtypeStruct((s, s, s), f32),
        in_specs=[pl.BlockSpec(memory_space=pltpu.VMEM)],
        out_specs=pl.BlockSpec(memory_space=pltpu.VMEM),
        scratch_shapes=[
            pltpu.VMEM((3, s, s), f32),
            pltpu.VMEM((3, s, s), f32),
            pltpu.SemaphoreType.DMA((3,)),
            pltpu.SemaphoreType.DMA((3,)),
        ],
        compiler_params=pltpu.CompilerParams(collective_id=0),
    )(u)


# device time: 7806 ns/iter; 1.1071x vs baseline; 1.1071x over previous
import jax
import jax.numpy as jnp
from jax import lax
from jax.experimental import pallas as pl
from jax.experimental.pallas import tpu as pltpu


def kernel(u):
    s = u.shape[0]
    f32 = jnp.float32

    def body(u_ref, out_ref, send_buf, halo_buf, send_sems, recv_sems):
        px = lax.axis_index("x")
        py = lax.axis_index("y")
        pz = lax.axis_index("z")

        nbrs = [(1 - px, py, pz), (px, 1 - py, pz), (px, py, 1 - pz)]
        barrier_sem = pltpu.get_barrier_semaphore()
        for nbr in nbrs:
            pl.semaphore_signal(
                barrier_sem, inc=1,
                device_id=nbr, device_id_type=pl.DeviceIdType.MESH,
            )

        uu = u_ref[...]

        send_buf[0] = jnp.where(px == 0, uu[s - 1, :, :], uu[0, :, :])
        send_buf[1] = jnp.where(py == 0, uu[:, s - 1, :], uu[:, 0, :])
        send_buf[2] = jnp.where(pz == 0, uu[:, :, s - 1], uu[:, :, 0])

        pl.semaphore_wait(barrier_sem, 3)

        rdmas = []
        for a, nbr in enumerate(nbrs):
            rdma = pltpu.make_async_remote_copy(
                src_ref=send_buf.at[a],
                dst_ref=halo_buf.at[a],
                send_sem=send_sems.at[a],
                recv_sem=recv_sems.at[a],
                device_id=nbr,
                device_id_type=pl.DeviceIdType.MESH,
            )
            rdma.start()
            rdmas.append(rdma)

        zero1 = jnp.zeros((1, s, s), f32)
        zero2 = jnp.zeros((s, 1, s), f32)
        zero3 = jnp.zeros((s, s, 1), f32)
        u_xm = jnp.concatenate([zero1, uu[:-1, :, :]], axis=0)
        u_xp = jnp.concatenate([uu[1:, :, :], zero1], axis=0)
        u_ym = jnp.concatenate([zero2, uu[:, :-1, :]], axis=1)
        u_yp = jnp.concatenate([uu[:, 1:, :], zero2], axis=1)
        u_zm = jnp.concatenate([zero3, uu[:, :, :-1]], axis=2)
        u_zp = jnp.concatenate([uu[:, :, 1:], zero3], axis=2)
        v = u_xm + u_xp + u_ym + u_yp + u_zm + u_zp - 6.0 * uu

        ii = lax.broadcasted_iota(jnp.int32, (s, s, s), 0)
        jj = lax.broadcasted_iota(jnp.int32, (s, s, s), 1)
        kk = lax.broadcasted_iota(jnp.int32, (s, s, s), 2)
        gi, gj, gk = ii + px * s, jj + py * s, kk + pz * s
        n = 2 * s - 1
        interior = (
            (gi > 0) & (gi < n) & (gj > 0) & (gj < n) & (gk > 0) & (gk < n)
        )
        mx = (ii == jnp.where(px == 0, s - 1, 0)).astype(f32)
        my = (jj == jnp.where(py == 0, s - 1, 0)).astype(f32)
        mz = (kk == jnp.where(pz == 0, s - 1, 0)).astype(f32)

        for rdma in rdmas:
            rdma.wait()

        v = v + mx * halo_buf[0][None, :, :]
        v = v + my * halo_buf[1][:, None, :]
        v = v + mz * halo_buf[2][:, :, None]
        out_ref[...] = jnp.where(interior, v, 0.0)

    return pl.pallas_call(
        body,
        out_shape=jax.ShapeDtypeStruct((s, s, s), f32),
        in_specs=[pl.BlockSpec(memory_space=pltpu.VMEM)],
        out_specs=pl.BlockSpec(memory_space=pltpu.VMEM),
        scratch_shapes=[
            pltpu.VMEM((3, s, s), f32),
            pltpu.VMEM((3, s, s), f32),
            pltpu.SemaphoreType.DMA((3,)),
            pltpu.SemaphoreType.DMA((3,)),
        ],
        compiler_params=pltpu.CompilerParams(collective_id=0),
    )(u)
